# baseline (speedup 1.0000x reference)
import functools

import jax
import jax.numpy as jnp
from jax import lax
from jax.experimental import pallas as pl
from jax.experimental.pallas import tpu as pltpu


def _se_dual_body(x_ref, w1_ref, w2_ref, o_ref, *, inv_hw):
    xa = x_ref[0, 0]
    xb = x_ref[1, 0]
    pa = jnp.sum(xa, axis=1, keepdims=True)
    pb = jnp.sum(xb, axis=1, keepdims=True)
    p = jnp.concatenate([pa, pb], axis=1) * jnp.float32(inv_hw)
    s1 = jnp.maximum(
        lax.dot_general(w1_ref[...], p, (((1,), (0,)), ((), ())),
                        preferred_element_type=jnp.float32),
        0.0,
    )
    z = lax.dot_general(w2_ref[...], s1, (((1,), (0,)), ((), ())),
                        preferred_element_type=jnp.float32)
    gate = jax.nn.sigmoid(z).astype(xa.dtype)
    o_ref[0, 0] = xa * gate[:, 0:1]
    o_ref[1, 0] = xb * gate[:, 1:2]


def kernel(x_nchw, w1, w2):
    N, C, H, W = x_nchw.shape
    HW = H * W
    Cr = w1.shape[0]
    Nh = N // 2

    x2 = x_nchw.reshape(2, Nh, C, HW)

    out2 = pl.pallas_call(
        functools.partial(_se_dual_body, inv_hw=1.0 / HW),
        out_shape=jax.ShapeDtypeStruct((2, Nh, C, HW), x_nchw.dtype),
        grid=(Nh,),
        in_specs=[
            pl.BlockSpec((2, 1, C, HW), lambda n: (0, n, 0, 0)),
            pl.BlockSpec((Cr, C), lambda n: (0, 0)),
            pl.BlockSpec((C, Cr), lambda n: (0, 0)),
        ],
        out_specs=pl.BlockSpec((2, 1, C, HW), lambda n: (0, n, 0, 0)),
        compiler_params=pltpu.CompilerParams(
            dimension_semantics=("parallel",),
            vmem_limit_bytes=64 * 1024 * 1024,
        ),
    )(x2, w1, w2)

    return out2.reshape(N, C, H, W)


# speedup vs baseline: 1.0067x; 1.0067x over previous
import functools

import jax
import jax.numpy as jnp
from jax import lax
from jax.experimental import pallas as pl
from jax.experimental.pallas import tpu as pltpu


def _se_dual2_body(xa_ref, xb_ref, w1_ref, w2_ref, o_ref, *, inv_hw):
    xa = xa_ref[0]
    xb = xb_ref[0]
    pa = jnp.sum(xa, axis=2)
    pb = jnp.sum(xb, axis=2)
    p = jnp.concatenate([pa, pb], axis=0) * jnp.float32(inv_hw)
    s1 = jnp.maximum(
        lax.dot_general(p, w1_ref[...], (((1,), (1,)), ((), ())),
                        preferred_element_type=jnp.float32),
        0.0,
    )
    z = lax.dot_general(s1, w2_ref[...], (((1,), (1,)), ((), ())),
                        preferred_element_type=jnp.float32)
    gate = jax.nn.sigmoid(z).astype(xa.dtype)
    o_ref[0] = xa * gate[0:2, :, None]
    o_ref[1] = xb * gate[2:4, :, None]


def kernel(x_nchw, w1, w2):
    N, C, H, W = x_nchw.shape
    HW = H * W
    Cr = w1.shape[0]
    Nh = N // 2

    x2 = x_nchw.reshape(2, Nh, C, HW)

    out2 = pl.pallas_call(
        functools.partial(_se_dual2_body, inv_hw=1.0 / HW),
        out_shape=jax.ShapeDtypeStruct((2, Nh, C, HW), x_nchw.dtype),
        grid=(Nh // 2,),
        in_specs=[
            pl.BlockSpec((1, 2, C, HW), lambda n: (0, n, 0, 0)),
            pl.BlockSpec((1, 2, C, HW), lambda n: (1, n, 0, 0)),
            pl.BlockSpec((Cr, C), lambda n: (0, 0)),
            pl.BlockSpec((C, Cr), lambda n: (0, 0)),
        ],
        out_specs=pl.BlockSpec((2, 2, C, HW), lambda n: (0, n, 0, 0)),
        compiler_params=pltpu.CompilerParams(
            dimension_semantics=("parallel",),
            vmem_limit_bytes=64 * 1024 * 1024,
        ),
    )(x2, x2, w1, w2)

    return out2.reshape(N, C, H, W)
